# baseline (speedup 1.0000x reference)
import jax
import jax.numpy as jnp
from jax.experimental import pallas as pl
from jax.experimental.pallas import tpu as pltpu

_N, _D = 16384, 1000
_K = 4
_R = 256
_S = _K * _R
_G = _N // _S


def _slice_loss_sum(x, t):
    col = jax.lax.broadcasted_iota(jnp.int32, (_R, _D), 1)
    is_t = col == t[:, None]
    v_y = jnp.sum(jnp.where(is_t, x, 0.0), axis=1)
    m_neq = jnp.max(jnp.where(is_t, -jnp.inf, x), axis=1)
    r = m_neq - v_y
    loss = jnp.clip(1.0 + r, 0.0, 1.0)
    return jnp.sum(loss)


def _ramp_body(tgt_ref, *refs):
    inp_refs = refs[:_K]
    out_ref = refs[_K]
    acc = jnp.float32(0.0)
    for k in range(_K):
        x = inp_refs[k][...]
        t = tgt_ref[0, 0, k * _R:(k + 1) * _R]
        acc += _slice_loss_sum(x, t)
    out_ref[...] = acc.reshape(1, 1, 1)


def kernel(inp, tgt):
    tgt3 = tgt.astype(jnp.int32).reshape(_G, 1, _S)
    in_specs = [pl.BlockSpec((1, 1, _S), lambda i: (i, 0, 0))]
    for k in range(_K):
        in_specs.append(
            pl.BlockSpec((_R, _D), lambda i, k=k: (_K * i + k, 0))
        )
    partials = pl.pallas_call(
        _ramp_body,
        grid=(_G,),
        in_specs=in_specs,
        out_specs=pl.BlockSpec((1, 1, 1), lambda i: (i, 0, 0)),
        out_shape=jax.ShapeDtypeStruct((_G, 1, 1), jnp.float32),
        compiler_params=pltpu.CompilerParams(
            dimension_semantics=("parallel",),
        ),
    )(tgt3, *([inp] * _K))
    return (jnp.sum(partials) / _N).reshape(1)
